# Initial kernel scaffold; baseline (speedup 1.0000x reference)
import jax
import jax.numpy as jnp
from jax.experimental import pallas as pl


def kernel(input, index, src):
    raise NotImplementedError("write your pallas kernel here")


# speedup vs baseline: 48.1007x; 48.1007x over previous
import functools

import jax
import jax.numpy as jnp
from jax import lax
from jax.experimental import pallas as pl
from jax.experimental.pallas import tpu as pltpu
from jax.experimental.pallas import tpu_sc as plsc

NC, NS = 2, 16
NW = NC * NS
L = 16


@functools.lru_cache(maxsize=None)
def _build(M, D, B, interpret=False):
    assert D % NW == 0
    cols_per_w = D // NW
    pair_chunk = min(B, 8192)
    assert B % pair_chunk == 0 and pair_chunk % L == 0
    mesh = plsc.VectorSubcoreMesh(
        core_axis_name="c", subcore_axis_name="s", num_cores=NC, num_subcores=NS
    )

    @functools.partial(
        pl.kernel,
        out_type=jax.ShapeDtypeStruct((D, M), jnp.float32),
        mesh=mesh,
        interpret=interpret,
        compiler_params=pltpu.CompilerParams(needs_layout_passes=False),
        scratch_types=[
            pltpu.VMEM((M,), jnp.float32),
            pltpu.VMEM((pair_chunk,), jnp.int32),
            pltpu.VMEM((pair_chunk,), jnp.float32),
        ],
    )
    def scatter_cols(inp_t, idx_t, src_t, out_t, col_v, idx_v, src_v):
        wid = lax.axis_index("s") * NC + lax.axis_index("c")
        for k in range(cols_per_w):
            col = k * NW + wid
            pltpu.sync_copy(inp_t.at[col], col_v)
            for cstart in range(0, B, pair_chunk):
                pltpu.sync_copy(idx_t.at[col, pl.ds(cstart, pair_chunk)], idx_v)
                pltpu.sync_copy(src_t.at[col, pl.ds(cstart, pair_chunk)], src_v)

                def body(i, carry):
                    base = i * L
                    idx16 = idx_v[pl.ds(base, L)]
                    val16 = src_v[pl.ds(base, L)]
                    plsc.addupdate_scatter(col_v, [idx16], val16)
                    return carry

                lax.fori_loop(0, pair_chunk // L, body, 0, unroll=8)
            pltpu.sync_copy(col_v, out_t.at[col])

    return scatter_cols


def kernel(input, index, src):
    M, D = input.shape
    B = src.shape[0]
    inp_t = input.T
    idx_t = index.astype(jnp.int32).T
    src_t = src.T
    out_t = _build(M, D, B)(inp_t, idx_t, src_t)
    return out_t.T
